# Initial kernel scaffold; baseline (speedup 1.0000x reference)
import jax
import jax.numpy as jnp
from jax.experimental import pallas as pl


def kernel(x, edge_index, W1, b1, W2, b2):
    raise NotImplementedError("write your pallas kernel here")


# speedup vs baseline: 21.5093x; 21.5093x over previous
import functools

import jax
import jax.numpy as jnp
from jax import lax
from jax.experimental import pallas as pl
from jax.experimental.pallas import tpu as pltpu
from jax.experimental.pallas import tpu_sc as plsc

N = 10000
D_IN = 128
DH = 16
NP = 10240
E_RAW = 320000
E_TOT = E_RAW + N
NC, NS, L = 2, 16, 16
NW = NC * NS
CHUNK = 128
EPT = 10368
NCHUNK = EPT // CHUNK
E_PAD = EPT * NW
RPS = NP // NS

_mesh = functools.partial(
    plsc.VectorSubcoreMesh, core_axis_name="c", subcore_axis_name="s"
)
_sc_params = pltpu.CompilerParams(use_tc_tiling_on_sc=False)


@functools.partial(
    pl.kernel,
    out_type=jax.ShapeDtypeStruct((NC, NP), jnp.float32),
    mesh=_mesh(),
    scratch_types=[
        pltpu.VMEM_SHARED((NP,), jnp.float32),
        pltpu.VMEM((CHUNK,), jnp.int32),
        pltpu.VMEM((CHUNK,), jnp.float32),
        pltpu.VMEM((RPS,), jnp.float32),
    ],
    compiler_params=_sc_params,
)
def _deg_kernel(dst_hbm, out_hbm, acc_sh, idx_v, ones_v, zero_v):
    c = lax.axis_index("c")
    s = lax.axis_index("s")
    wid = s * NC + c

    def fill_ones(i, carry):
        ones_v[pl.ds(i * L, L)] = jnp.full((L,), 1.0, jnp.float32)
        return carry

    lax.fori_loop(0, CHUNK // L, fill_ones, 0)

    def fill_zero(i, carry):
        zero_v[pl.ds(i * L, L)] = jnp.zeros((L,), jnp.float32)
        return carry

    lax.fori_loop(0, RPS // L, fill_zero, 0)

    pltpu.sync_copy(zero_v, acc_sh.at[pl.ds(s * RPS, RPS)])
    plsc.subcore_barrier()

    def body(j, carry):
        base = wid * EPT + j * CHUNK
        pltpu.sync_copy(dst_hbm.at[pl.ds(base, CHUNK)], idx_v)
        pltpu.sync_copy(ones_v, acc_sh.at[idx_v], add=True)
        return carry

    lax.fori_loop(0, NCHUNK, body, 0)

    plsc.subcore_barrier()
    pltpu.sync_copy(
        acc_sh.at[pl.ds(s * RPS, RPS)], out_hbm.at[c, pl.ds(s * RPS, RPS)]
    )


@functools.partial(
    pl.kernel,
    out_type=jax.ShapeDtypeStruct((NC, NP, DH), jnp.float32),
    mesh=_mesh(),
    scratch_types=[
        pltpu.VMEM_SHARED((NP, DH), jnp.float32),
        pltpu.VMEM((CHUNK,), jnp.int32),
        pltpu.VMEM((CHUNK,), jnp.int32),
        pltpu.VMEM((CHUNK, DH), jnp.float32),
        pltpu.VMEM((RPS, DH), jnp.float32),
        pltpu.SemaphoreType.DMA,
    ],
    compiler_params=_sc_params,
)
def _agg_kernel(feat_hbm, src_hbm, dst_hbm, out_hbm,
                acc_sh, sidx_v, didx_v, rows_v, zero_v, sem):
    c = lax.axis_index("c")
    s = lax.axis_index("s")
    wid = s * NC + c

    def fill_zero(i, carry):
        zero_v[i, :] = jnp.zeros((L,), jnp.float32)
        return carry

    lax.fori_loop(0, RPS, fill_zero, 0)

    pltpu.sync_copy(zero_v, acc_sh.at[pl.ds(s * RPS, RPS)])
    plsc.subcore_barrier()

    def body(j, carry):
        base = wid * EPT + j * CHUNK
        pltpu.sync_copy(src_hbm.at[pl.ds(base, CHUNK)], sidx_v)
        pltpu.sync_copy(dst_hbm.at[pl.ds(base, CHUNK)], didx_v)
        pltpu.async_copy(feat_hbm.at[sidx_v], rows_v, sem).wait()
        pltpu.sync_copy(rows_v, acc_sh.at[didx_v], add=True)
        return carry

    lax.fori_loop(0, NCHUNK, body, 0)

    plsc.subcore_barrier()
    pltpu.sync_copy(
        acc_sh.at[pl.ds(s * RPS, RPS)], out_hbm.at[c, pl.ds(s * RPS, RPS)]
    )


def _tca_body(x_ref, w1_ref, deg_ref, hs_ref, dis_ref):
    deg = deg_ref[0] + deg_ref[1]
    dis = jnp.where(deg > 0, lax.rsqrt(jnp.maximum(deg, 1e-12)), 0.0)
    dis_ref[...] = dis
    h = jnp.dot(x_ref[...], w1_ref[...], preferred_element_type=jnp.float32)
    h = jnp.concatenate([h, jnp.zeros((NP - N, DH), jnp.float32)], axis=0)
    hs_ref[...] = h * dis[:, None]


def _tcb_body(s1_ref, dis_ref, b1_ref, gs_ref):
    s = s1_ref[0] + s1_ref[1]
    dis = dis_ref[...]
    x2 = jnp.maximum(s * dis[:, None] + b1_ref[...][None, :], 0.0)
    gs_ref[...] = x2 * dis[:, None]


def _tcc_body(s2_ref, dis_ref, w2_ref, b2_ref, out_ref):
    s = s2_ref[0] + s2_ref[1]
    dis = dis_ref[...]
    z = jnp.dot(s * dis[:, None], w2_ref[...],
                preferred_element_type=jnp.float32) + b2_ref[...][None, :]
    m = jnp.max(z, axis=1, keepdims=True)
    lse = m + jnp.log(jnp.sum(jnp.exp(z - m), axis=1, keepdims=True))
    out_ref[...] = z - lse


def kernel(x, edge_index, W1, b1, W2, b2):
    loop = jnp.arange(N, dtype=jnp.int32)
    pad = jnp.full((E_PAD - E_TOT,), N, jnp.int32)
    src = jnp.concatenate([edge_index[0], loop, pad])
    dst = jnp.concatenate([edge_index[1], loop, pad])

    deg2 = _deg_kernel(dst)

    hs, dis = pl.pallas_call(
        _tca_body,
        out_shape=(
            jax.ShapeDtypeStruct((NP, DH), jnp.float32),
            jax.ShapeDtypeStruct((NP,), jnp.float32),
        ),
    )(x, W1, deg2)

    s1 = _agg_kernel(hs, src, dst)

    gs = pl.pallas_call(
        _tcb_body,
        out_shape=jax.ShapeDtypeStruct((NP, DH), jnp.float32),
    )(s1, dis, b1)

    s2 = _agg_kernel(gs, src, dst)

    out = pl.pallas_call(
        _tcc_body,
        out_shape=jax.ShapeDtypeStruct((NP, 2), jnp.float32),
    )(s2, dis, W2, b2)

    return out[:N]


# speedup vs baseline: 31.2929x; 1.4549x over previous
import functools

import jax
import jax.numpy as jnp
from jax import lax
from jax.experimental import pallas as pl
from jax.experimental.pallas import tpu as pltpu
from jax.experimental.pallas import tpu_sc as plsc

N = 10000
D_IN = 128
DH = 16
NP = 10240
E_RAW = 320000
E_TOT = E_RAW + N
NC, NS, L = 2, 16, 16
NW = NC * NS
CHUNK = 128
NBUF = 4
NCHUNK = 84
NGRP = NCHUNK // NBUF
EPT = NCHUNK * CHUNK
E_PAD = EPT * NW
RPS = NP // NS

_mesh = functools.partial(
    plsc.VectorSubcoreMesh, core_axis_name="c", subcore_axis_name="s"
)
_sc_params = pltpu.CompilerParams(use_tc_tiling_on_sc=False)


@functools.partial(
    pl.kernel,
    out_type=jax.ShapeDtypeStruct((NC, NP), jnp.float32),
    mesh=_mesh(),
    scratch_types=[
        pltpu.VMEM_SHARED((NP,), jnp.float32),
        pltpu.VMEM((NCHUNK, CHUNK), jnp.int32),
        pltpu.VMEM((CHUNK,), jnp.float32),
        pltpu.VMEM((RPS,), jnp.float32),
        pltpu.SemaphoreType.DMA((NBUF,)),
        pltpu.SemaphoreType.DMA,
    ],
    compiler_params=_sc_params,
)
def _deg_kernel(dst_hbm, out_hbm, acc_sh, dst_v, ones_v, zero_v, ssem, isem):
    c = lax.axis_index("c")
    s = lax.axis_index("s")
    wid = s * NC + c

    idx_cp = pltpu.async_copy(dst_hbm.at[wid], dst_v, isem)

    def fill_ones(i, carry):
        ones_v[pl.ds(i * L, L)] = jnp.full((L,), 1.0, jnp.float32)
        return carry

    lax.fori_loop(0, CHUNK // L, fill_ones, 0)

    def fill_zero(i, carry):
        zero_v[pl.ds(i * L, L)] = jnp.zeros((L,), jnp.float32)
        return carry

    lax.fori_loop(0, RPS // L, fill_zero, 0)

    pltpu.sync_copy(zero_v, acc_sh.at[pl.ds(s * RPS, RPS)])
    idx_cp.wait()
    plsc.subcore_barrier()

    def group(g, carry):
        for b in range(NBUF):
            j = g * NBUF + b
            pltpu.async_copy(ones_v, acc_sh.at[dst_v.at[j]], ssem.at[b],
                             add=True)
        for b in range(NBUF):
            pltpu.make_async_copy(ones_v, acc_sh.at[dst_v.at[b]],
                                  ssem.at[b]).wait()
        return carry

    lax.fori_loop(0, NGRP, group, 0)

    plsc.subcore_barrier()
    pltpu.sync_copy(
        acc_sh.at[pl.ds(s * RPS, RPS)], out_hbm.at[c, pl.ds(s * RPS, RPS)]
    )


@functools.partial(
    pl.kernel,
    out_type=jax.ShapeDtypeStruct((NC, NP, DH), jnp.float32),
    mesh=_mesh(),
    scratch_types=[
        pltpu.VMEM_SHARED((NP, DH), jnp.float32),
        pltpu.VMEM((NCHUNK, CHUNK), jnp.int32),
        pltpu.VMEM((NCHUNK, CHUNK), jnp.int32),
        pltpu.VMEM((NBUF, CHUNK, DH), jnp.float32),
        pltpu.VMEM((RPS, DH), jnp.float32),
        pltpu.SemaphoreType.DMA((NBUF,)),
        pltpu.SemaphoreType.DMA((NBUF,)),
        pltpu.SemaphoreType.DMA,
    ],
    compiler_params=_sc_params,
)
def _agg_kernel(feat_hbm, src_hbm, dst_hbm, out_hbm,
                acc_sh, src_v, dst_v, rows_v, zero_v, gsem, ssem, isem):
    c = lax.axis_index("c")
    s = lax.axis_index("s")
    wid = s * NC + c

    cp_s = pltpu.async_copy(src_hbm.at[wid], src_v, isem)
    cp_d = pltpu.async_copy(dst_hbm.at[wid], dst_v, isem)

    def fill_zero(i, carry):
        zero_v[i, :] = jnp.zeros((L,), jnp.float32)
        return carry

    lax.fori_loop(0, RPS, fill_zero, 0)
    pltpu.sync_copy(zero_v, acc_sh.at[pl.ds(s * RPS, RPS)])
    cp_s.wait()
    cp_d.wait()
    plsc.subcore_barrier()

    for b in range(NBUF):
        pltpu.async_copy(feat_hbm.at[src_v.at[b]], rows_v.at[b], gsem.at[b])

    def group(g, carry):
        for b in range(NBUF):
            j = g * NBUF + b
            pltpu.make_async_copy(feat_hbm.at[src_v.at[b]], rows_v.at[b],
                                  gsem.at[b]).wait()
            pltpu.async_copy(rows_v.at[b], acc_sh.at[dst_v.at[j]],
                             ssem.at[b], add=True)
        for b in range(NBUF):
            nxt = g * NBUF + b + NBUF
            pltpu.make_async_copy(rows_v.at[b], acc_sh.at[dst_v.at[b]],
                                  ssem.at[b]).wait()

            @pl.when(nxt < NCHUNK)
            def _():
                pltpu.async_copy(feat_hbm.at[src_v.at[nxt]], rows_v.at[b],
                                 gsem.at[b])

        return carry

    lax.fori_loop(0, NGRP, group, 0)

    plsc.subcore_barrier()
    pltpu.sync_copy(
        acc_sh.at[pl.ds(s * RPS, RPS)], out_hbm.at[c, pl.ds(s * RPS, RPS)]
    )


def _tca_body(x_ref, w1_ref, deg_ref, hs_ref, dis_ref):
    deg = deg_ref[0] + deg_ref[1]
    dis = jnp.where(deg > 0, lax.rsqrt(jnp.maximum(deg, 1e-12)), 0.0)
    dis_ref[...] = dis
    h = jnp.dot(x_ref[...], w1_ref[...], preferred_element_type=jnp.float32)
    h = jnp.concatenate([h, jnp.zeros((NP - N, DH), jnp.float32)], axis=0)
    hs_ref[...] = h * dis[:, None]


def _tcb_body(s1_ref, dis_ref, b1_ref, gs_ref):
    s = s1_ref[0] + s1_ref[1]
    dis = dis_ref[...]
    x2 = jnp.maximum(s * dis[:, None] + b1_ref[...][None, :], 0.0)
    gs_ref[...] = x2 * dis[:, None]


def _tcc_body(s2_ref, dis_ref, w2_ref, b2_ref, out_ref):
    s = s2_ref[0] + s2_ref[1]
    dis = dis_ref[...]
    z = jnp.dot(s * dis[:, None], w2_ref[...],
                preferred_element_type=jnp.float32) + b2_ref[...][None, :]
    m = jnp.max(z, axis=1, keepdims=True)
    lse = m + jnp.log(jnp.sum(jnp.exp(z - m), axis=1, keepdims=True))
    out_ref[...] = z - lse


def kernel(x, edge_index, W1, b1, W2, b2):
    loop = jnp.arange(N, dtype=jnp.int32)
    pad = jnp.full((E_PAD - E_TOT,), N, jnp.int32)
    src = jnp.concatenate([edge_index[0], loop, pad]).reshape(NW, NCHUNK, CHUNK)
    dst = jnp.concatenate([edge_index[1], loop, pad]).reshape(NW, NCHUNK, CHUNK)

    deg2 = _deg_kernel(dst)

    hs, dis = pl.pallas_call(
        _tca_body,
        out_shape=(
            jax.ShapeDtypeStruct((NP, DH), jnp.float32),
            jax.ShapeDtypeStruct((NP,), jnp.float32),
        ),
    )(x, W1, deg2)

    s1 = _agg_kernel(hs, src, dst)

    gs = pl.pallas_call(
        _tcb_body,
        out_shape=jax.ShapeDtypeStruct((NP, DH), jnp.float32),
    )(s1, dis, b1)

    s2 = _agg_kernel(gs, src, dst)

    out = pl.pallas_call(
        _tcc_body,
        out_shape=jax.ShapeDtypeStruct((NP, 2), jnp.float32),
    )(s2, dis, W2, b2)

    return out[:N]


# speedup vs baseline: 45.8234x; 1.4643x over previous
import functools

import jax
import jax.numpy as jnp
from jax import lax
from jax.experimental import pallas as pl
from jax.experimental.pallas import tpu as pltpu
from jax.experimental.pallas import tpu_sc as plsc

N = 10000
D_IN = 128
DH = 16
DO = 8
NP = 10240
NZ = NP - N
E_RAW = 320000
E_TOT = E_RAW + N
NC, NS, L = 2, 16, 16
CHUNK = 128
NBUF = 4
CH0 = 104
CH1 = 60
CHMAX = max(CH0, CH1)
TOTAL_CH = NS * (CH0 + CH1)
ROWS_PAD = TOTAL_CH + CHMAX - min(CH0, CH1)
E_PAD = ROWS_PAD * CHUNK
RPS = NP // NS

_mesh = functools.partial(
    plsc.VectorSubcoreMesh, core_axis_name="c", subcore_axis_name="s"
)
_sc_params = pltpu.CompilerParams(use_tc_tiling_on_sc=False)


def _tile_plan(c, s):
    start = jnp.where(c == 0, s * CH0, NS * CH0 + s * CH1)
    nch = jnp.where(c == 0, CH0, CH1)
    ngrp = jnp.where(c == 0, CH0 // NBUF, CH1 // NBUF)
    return start, nch, ngrp


@functools.partial(
    pl.kernel,
    out_type=jax.ShapeDtypeStruct((NC, NP), jnp.float32),
    mesh=_mesh(),
    scratch_types=[
        pltpu.VMEM_SHARED((NP,), jnp.float32),
        pltpu.VMEM((CHMAX, CHUNK), jnp.int32),
        pltpu.VMEM((CHUNK,), jnp.float32),
        pltpu.VMEM((RPS,), jnp.float32),
        pltpu.SemaphoreType.DMA((NBUF,)),
        pltpu.SemaphoreType.DMA,
    ],
    compiler_params=_sc_params,
)
def _deg_kernel(dst_hbm, out_hbm, acc_sh, dst_v, ones_v, zero_v, ssem, isem):
    c = lax.axis_index("c")
    s = lax.axis_index("s")
    start, nch, ngrp = _tile_plan(c, s)

    idx_cp = pltpu.async_copy(dst_hbm.at[pl.ds(start, CHMAX)], dst_v, isem)

    def fill_ones(i, carry):
        ones_v[pl.ds(i * L, L)] = jnp.full((L,), 1.0, jnp.float32)
        return carry

    lax.fori_loop(0, CHUNK // L, fill_ones, 0)

    def fill_zero(i, carry):
        zero_v[pl.ds(i * L, L)] = jnp.zeros((L,), jnp.float32)
        return carry

    lax.fori_loop(0, RPS // L, fill_zero, 0)

    pltpu.sync_copy(zero_v, acc_sh.at[pl.ds(s * RPS, RPS)])
    idx_cp.wait()
    plsc.subcore_barrier()

    def group(g, carry):
        for b in range(NBUF):
            j = g * NBUF + b
            pltpu.async_copy(ones_v, acc_sh.at[dst_v.at[j]], ssem.at[b],
                             add=True)
        for b in range(NBUF):
            pltpu.make_async_copy(ones_v, acc_sh.at[dst_v.at[b]],
                                  ssem.at[b]).wait()
        return carry

    lax.fori_loop(0, ngrp, group, 0)

    plsc.subcore_barrier()
    pltpu.sync_copy(
        acc_sh.at[pl.ds(s * RPS, RPS)], out_hbm.at[c, pl.ds(s * RPS, RPS)]
    )


def _agg_body(D, feat_hbm, src_hbm, dst_hbm, out_hbm,
              acc_sh, src_v, dst_v, rows_v, gsem, ssem, isem):
    c = lax.axis_index("c")
    s = lax.axis_index("s")
    start, nch, ngrp = _tile_plan(c, s)

    cp_s = pltpu.async_copy(src_hbm.at[pl.ds(start, CHMAX)], src_v, isem)
    cp_d = pltpu.async_copy(dst_hbm.at[pl.ds(start, CHMAX)], dst_v, isem)

    base = s * RPS
    pltpu.sync_copy(feat_hbm.at[pl.ds(N, NZ)], acc_sh.at[pl.ds(base, NZ)])
    pltpu.sync_copy(feat_hbm.at[pl.ds(N, NZ)],
                    acc_sh.at[pl.ds(base + NZ, NZ)])
    pltpu.sync_copy(feat_hbm.at[pl.ds(N, RPS - 2 * NZ)],
                    acc_sh.at[pl.ds(base + 2 * NZ, RPS - 2 * NZ)])
    cp_s.wait()
    cp_d.wait()
    plsc.subcore_barrier()

    for b in range(NBUF):
        pltpu.async_copy(feat_hbm.at[src_v.at[b]], rows_v.at[b], gsem.at[b])

    def group(g, carry):
        for b in range(NBUF):
            j = g * NBUF + b
            pltpu.make_async_copy(feat_hbm.at[src_v.at[b]], rows_v.at[b],
                                  gsem.at[b]).wait()
            pltpu.async_copy(rows_v.at[b], acc_sh.at[dst_v.at[j]],
                             ssem.at[b], add=True)
        for b in range(NBUF):
            nxt = g * NBUF + b + NBUF
            pltpu.make_async_copy(rows_v.at[b], acc_sh.at[dst_v.at[b]],
                                  ssem.at[b]).wait()

            @pl.when(nxt < nch)
            def _():
                pltpu.async_copy(feat_hbm.at[src_v.at[nxt]], rows_v.at[b],
                                 gsem.at[b])

        return carry

    lax.fori_loop(0, ngrp, group, 0)

    plsc.subcore_barrier()
    pltpu.sync_copy(
        acc_sh.at[pl.ds(s * RPS, RPS)], out_hbm.at[c, pl.ds(s * RPS, RPS)]
    )


def _make_agg(D):
    return functools.partial(
        pl.kernel,
        out_type=jax.ShapeDtypeStruct((NC, NP, D), jnp.float32),
        mesh=_mesh(),
        scratch_types=[
            pltpu.VMEM_SHARED((NP, D), jnp.float32),
            pltpu.VMEM((CHMAX, CHUNK), jnp.int32),
            pltpu.VMEM((CHMAX, CHUNK), jnp.int32),
            pltpu.VMEM((NBUF, CHUNK, D), jnp.float32),
            pltpu.SemaphoreType.DMA((NBUF,)),
            pltpu.SemaphoreType.DMA((NBUF,)),
            pltpu.SemaphoreType.DMA,
        ],
        compiler_params=_sc_params,
    )(functools.partial(_agg_body, D))


_agg16 = _make_agg(DH)
_agg4 = _make_agg(DO)


def _tca_body(x_ref, w1_ref, deg_ref, hs_ref, dis_ref):
    deg = deg_ref[0] + deg_ref[1]
    dis = jnp.where(deg > 0, lax.rsqrt(jnp.maximum(deg, 1e-12)), 0.0)
    dis_ref[...] = dis
    h = jnp.dot(x_ref[...], w1_ref[...], preferred_element_type=jnp.float32)
    h = jnp.concatenate([h, jnp.zeros((NP - N, DH), jnp.float32)], axis=0)
    hs_ref[...] = h * dis[:, None]


def _tcb_body(s1_ref, dis_ref, b1_ref, w2_ref, gs_ref):
    s = s1_ref[0] + s1_ref[1]
    dis = dis_ref[...]
    x2 = jnp.maximum(s * dis[:, None] + b1_ref[...][None, :], 0.0)
    g = jnp.dot(x2, w2_ref[...], preferred_element_type=jnp.float32)
    g = g * dis[:, None]
    row = lax.broadcasted_iota(jnp.int32, (NP, DO), 0)
    gs_ref[...] = jnp.where(row < N, g, 0.0)


def _tcc_body(s2_ref, dis_ref, b2_ref, out_ref):
    su = s2_ref[0] + s2_ref[1]
    dis = dis_ref[...]
    z = su[:, :2] * dis[:, None] + b2_ref[...][None, :]
    m = jnp.max(z, axis=1, keepdims=True)
    lse = m + jnp.log(jnp.sum(jnp.exp(z - m), axis=1, keepdims=True))
    out_ref[...] = z - lse


def kernel(x, edge_index, W1, b1, W2, b2):
    loop = jnp.arange(N, dtype=jnp.int32)
    pad = jnp.full((E_PAD - E_TOT,), N, jnp.int32)
    src = jnp.concatenate([edge_index[0], loop, pad]).reshape(ROWS_PAD, CHUNK)
    dst = jnp.concatenate([edge_index[1], loop, pad]).reshape(ROWS_PAD, CHUNK)
    W2p = jnp.pad(W2, ((0, 0), (0, DO - 2)))

    deg2 = _deg_kernel(dst)

    hs, dis = pl.pallas_call(
        _tca_body,
        out_shape=(
            jax.ShapeDtypeStruct((NP, DH), jnp.float32),
            jax.ShapeDtypeStruct((NP,), jnp.float32),
        ),
    )(x, W1, deg2)

    s1 = _agg16(hs, src, dst)

    gs = pl.pallas_call(
        _tcb_body,
        out_shape=jax.ShapeDtypeStruct((NP, DO), jnp.float32),
    )(s1, dis, b1, W2p)

    s2 = _agg4(gs, src, dst)

    out = pl.pallas_call(
        _tcc_body,
        out_shape=jax.ShapeDtypeStruct((NP, 2), jnp.float32),
    )(s2, dis, b2)

    return out[:N]
